# baseline (speedup 1.0000x reference)
import jax
import jax.numpy as jnp
import numpy as np
from jax import lax
from jax.experimental import pallas as pl
from jax.experimental.pallas import tpu as pltpu
from jax.experimental.pallas import tpu_sc as plsc

_RP = 0.01
_NC = 2
_NS = 16
_NW = _NC * _NS
_L = 16

_EPS32 = float(np.finfo(np.float32).eps)
_S = float(np.float32((1.0 - _EPS32) / 128.0))
_INV_S = float(np.float32(1.0) / np.float32(_S))


def _prep_body(w_ref, knots_ref, m_ref):
    w = w_ref[...]
    R, N = w.shape

    wl = jnp.concatenate([w[:, :1], w[:, :-1]], axis=1)
    wr = jnp.concatenate([w[:, 1:], w[:, -1:]], axis=1)
    w2 = 0.5 * (jnp.maximum(wl, w) + jnp.maximum(w, wr)) + _RP

    ws = jnp.sum(w2, axis=1, keepdims=True)
    pad = jnp.maximum(0.0, 1e-5 - ws)
    pdf = (w2 + pad / N) / (ws + pad)

    x = pdf
    d = 1
    while d < N:
        x = x + jnp.concatenate([jnp.zeros((R, d), x.dtype), x[:, :-d]], axis=1)
        d *= 2
    knots = jnp.concatenate(
        [jnp.zeros((R, 1), x.dtype), jnp.minimum(x[:, : N - 1], 1.0),
         jnp.ones((R, 1), x.dtype)], axis=1)
    knots_ref[...] = knots

    kn = knots[:, 1:]
    xm = kn * _INV_S
    m0 = xm.astype(jnp.int32)
    m0 = m0 + (m0.astype(jnp.float32) < xm).astype(jnp.int32)
    m = (m0
         - ((m0 - 1).astype(jnp.float32) * _S >= kn).astype(jnp.int32)
         + (m0.astype(jnp.float32) * _S < kn).astype(jnp.int32))
    m_ref[...] = m


def _sc_body(knots3, m3, t3, out3, kv, mv, tv, ov, hist):
    cid = lax.axis_index("c")
    sid = lax.axis_index("s")
    wid = sid * _NC + cid

    pltpu.sync_copy(knots3.at[wid], kv)
    pltpu.sync_copy(m3.at[wid], mv)
    pltpu.sync_copy(t3.at[wid], tv)

    iota = lax.iota(jnp.int32, _L)
    ones_i = jnp.ones((_L,), jnp.int32)
    zeros_i = jnp.zeros((_L,), jnp.int32)

    def _zero(j, c):
        hist[pl.ds(j * _L, _L)] = zeros_i
        return c

    lax.fori_loop(0, 130, _zero, 0, unroll=5)

    for g in range(128 // _L):
        co = g * _L
        col128 = (iota + co) * 128
        col129 = (iota + co) * 129

        def _scatter(k, c):
            mk = plsc.load_gather(mv, [col128 + k])
            plsc.addupdate_scatter(hist, [mk * _L + iota], ones_i)
            return c

        lax.fori_loop(0, 128, _scatter, 0, unroll=4)

        def _search(j, cnt):
            h = hist[pl.ds(j * _L, _L)]
            hist[pl.ds(j * _L, _L)] = zeros_i
            cnt = cnt + h
            ks = cnt - 1
            fk = col129 + ks
            c0 = plsc.load_gather(kv, [fk])
            c1 = plsc.load_gather(kv, [fk + 1])
            b0 = plsc.load_gather(tv, [fk])
            b1 = plsc.load_gather(tv, [fk + 1])
            uj = j.astype(jnp.float32) * _S
            t = jnp.clip((uj - c0) / (c1 - c0), 0.0, 1.0)
            plsc.store_scatter(ov, [col129 + j], b0 + t * (b1 - b0))
            return cnt

        lax.fori_loop(0, 129, _search, ones_i, unroll=3)

    pltpu.sync_copy(ov, out3.at[wid])


_sc_search = pl.kernel(
    _sc_body,
    out_type=jax.ShapeDtypeStruct((_NW, 129 * 128), jnp.float32),
    mesh=plsc.VectorSubcoreMesh(core_axis_name="c", subcore_axis_name="s",
                                num_cores=_NC, num_subcores=_NS),
    compiler_params=pltpu.CompilerParams(needs_layout_passes=False),
    scratch_types=[
        pltpu.VMEM((129 * 128,), jnp.float32),
        pltpu.VMEM((128 * 128,), jnp.int32),
        pltpu.VMEM((129 * 128,), jnp.float32),
        pltpu.VMEM((129 * 128,), jnp.float32),
        pltpu.VMEM((130 * _L,), jnp.int32),
    ],
)


def _cast_body(rays_ref, radii_ref, newt_ref, means_ref, covs_ref):
    new_t = newt_ref[...]
    N = new_t.shape[1] - 1
    t0 = new_t[:, :N]
    t1 = new_t[:, 1:]
    t_mean = (t0 + t1) / 2
    t_var = (t1 - t0) ** 2 / 12
    radii = radii_ref[...]
    r_var = radii ** 2 / 4
    rays = rays_ref[...]
    o = rays[:, 0:3]
    dvec = rays[:, 3:6]
    dmag = jnp.maximum(1e-10, jnp.sum(dvec * dvec, axis=1, keepdims=True))
    d2 = dvec * dvec
    null = 1.0 - d2 / dmag
    means_ref[...] = jnp.concatenate(
        [dvec[:, c:c + 1] * t_mean + o[:, c:c + 1] for c in range(3)], axis=1)
    covs_ref[...] = jnp.concatenate(
        [t_var * d2[:, c:c + 1] + r_var * null[:, c:c + 1] for c in range(3)],
        axis=1)


def kernel(rays, radii, weights, t_vals):
    B = rays.shape[0]
    N = weights.shape[1]
    w2d = jnp.squeeze(weights, axis=-1)

    R1 = 512
    knots, m = pl.pallas_call(
        _prep_body,
        grid=(B // R1,),
        in_specs=[pl.BlockSpec((R1, N), lambda i: (i, 0))],
        out_specs=[pl.BlockSpec((R1, N + 1), lambda i: (i, 0)),
                   pl.BlockSpec((R1, N), lambda i: (i, 0))],
        out_shape=[jax.ShapeDtypeStruct((B, N + 1), jnp.float32),
                   jax.ShapeDtypeStruct((B, N), jnp.int32)],
    )(w2d)

    rpw = B // _NW
    knots3 = knots.reshape(_NW, rpw * (N + 1))
    t3 = t_vals.reshape(_NW, rpw * (N + 1))
    m3 = m.reshape(_NW, rpw * N)

    out3 = _sc_search(knots3, m3, t3)
    new_t = out3.reshape(B, N + 1)

    R2 = 512
    means_t, covs_t = pl.pallas_call(
        _cast_body,
        grid=(B // R2,),
        in_specs=[
            pl.BlockSpec((R2, 6), lambda i: (i, 0)),
            pl.BlockSpec((R2, 1), lambda i: (i, 0)),
            pl.BlockSpec((R2, N + 1), lambda i: (i, 0)),
        ],
        out_specs=[pl.BlockSpec((R2, 3 * N), lambda i: (i, 0)),
                   pl.BlockSpec((R2, 3 * N), lambda i: (i, 0))],
        out_shape=[jax.ShapeDtypeStruct((B, 3 * N), jnp.float32),
                   jax.ShapeDtypeStruct((B, 3 * N), jnp.float32)],
    )(rays, radii, new_t)

    means = jnp.transpose(means_t.reshape(B, 3, N), (0, 2, 1))
    covs = jnp.transpose(covs_t.reshape(B, 3, N), (0, 2, 1))
    return (new_t, means, covs)


# speedup vs baseline: 1.0755x; 1.0755x over previous
import jax
import jax.numpy as jnp
import numpy as np
from jax import lax
from jax.experimental import pallas as pl
from jax.experimental.pallas import tpu as pltpu
from jax.experimental.pallas import tpu_sc as plsc

_RP = 0.01
_NC = 2
_NS = 16
_NW = _NC * _NS
_L = 16

_EPS32 = float(np.finfo(np.float32).eps)
_S = float(np.float32((1.0 - _EPS32) / 128.0))
_INV_S = float(np.float32(1.0) / np.float32(_S))


def _prep_body(w_ref, knots_ref, m_ref):
    w = w_ref[...]
    R, N = w.shape

    wl = jnp.concatenate([w[:, :1], w[:, :-1]], axis=1)
    wr = jnp.concatenate([w[:, 1:], w[:, -1:]], axis=1)
    w2 = 0.5 * (jnp.maximum(wl, w) + jnp.maximum(w, wr)) + _RP

    ws = jnp.sum(w2, axis=1, keepdims=True)
    pad = jnp.maximum(0.0, 1e-5 - ws)
    pdf = (w2 + pad / N) / (ws + pad)

    x = pdf
    d = 1
    while d < N:
        x = x + jnp.concatenate([jnp.zeros((R, d), x.dtype), x[:, :-d]], axis=1)
        d *= 2
    knots = jnp.concatenate(
        [jnp.zeros((R, 1), x.dtype), jnp.minimum(x[:, : N - 1], 1.0),
         jnp.ones((R, 1), x.dtype)], axis=1)
    knots_ref[...] = knots

    kn = knots[:, 1:]
    xm = kn * _INV_S
    m0 = xm.astype(jnp.int32)
    m0 = m0 + (m0.astype(jnp.float32) < xm).astype(jnp.int32)
    m = (m0
         - ((m0 - 1).astype(jnp.float32) * _S >= kn).astype(jnp.int32)
         + (m0.astype(jnp.float32) * _S < kn).astype(jnp.int32))
    m_ref[...] = m


def _sc_body(knots3, m3, t3, out3, kv, mv, tv, ov, hist):
    cid = lax.axis_index("c")
    sid = lax.axis_index("s")
    wid = sid * _NC + cid

    pltpu.sync_copy(knots3.at[wid], kv)
    pltpu.sync_copy(m3.at[wid], mv)
    pltpu.sync_copy(t3.at[wid], tv)

    iota = lax.iota(jnp.int32, _L)
    ones_i = jnp.ones((_L,), jnp.int32)
    zeros_i = jnp.zeros((_L,), jnp.int32)

    def _zero(j, c):
        hist[pl.ds(j * _L, _L)] = zeros_i
        return c

    lax.fori_loop(0, 130, _zero, 0, unroll=5)

    for g in range(128 // _L):
        co = g * _L
        col129 = (iota + co) * 129

        def _scatter(k, c):
            mk = mv[pl.ds(k * 128 + co, _L)]
            plsc.addupdate_scatter(hist, [mk * _L + iota], ones_i)
            return c

        lax.fori_loop(0, 128, _scatter, 0, unroll=4)

        def _search(j, cnt):
            h = hist[pl.ds(j * _L, _L)]
            hist[pl.ds(j * _L, _L)] = zeros_i
            cnt = cnt + h
            ks = cnt - 1
            fk = col129 + ks
            c0 = plsc.load_gather(kv, [fk])
            c1 = plsc.load_gather(kv, [fk + 1])
            b0 = plsc.load_gather(tv, [fk])
            b1 = plsc.load_gather(tv, [fk + 1])
            uj = j.astype(jnp.float32) * _S
            t = jnp.clip((uj - c0) / (c1 - c0), 0.0, 1.0)
            ov[pl.ds(j * 128 + co, _L)] = b0 + t * (b1 - b0)
            return cnt

        lax.fori_loop(0, 129, _search, ones_i, unroll=3)

    pltpu.sync_copy(ov, out3.at[wid])


_sc_search = pl.kernel(
    _sc_body,
    out_type=jax.ShapeDtypeStruct((_NW, 129 * 128), jnp.float32),
    mesh=plsc.VectorSubcoreMesh(core_axis_name="c", subcore_axis_name="s",
                                num_cores=_NC, num_subcores=_NS),
    compiler_params=pltpu.CompilerParams(needs_layout_passes=False),
    scratch_types=[
        pltpu.VMEM((129 * 128,), jnp.float32),
        pltpu.VMEM((128 * 128,), jnp.int32),
        pltpu.VMEM((129 * 128,), jnp.float32),
        pltpu.VMEM((129 * 128,), jnp.float32),
        pltpu.VMEM((130 * _L,), jnp.int32),
    ],
)


def _cast_body(rays_ref, radii_ref, newt_ref, means_ref, covs_ref):
    new_t = newt_ref[...]
    N = new_t.shape[1] - 1
    t0 = new_t[:, :N]
    t1 = new_t[:, 1:]
    t_mean = (t0 + t1) / 2
    t_var = (t1 - t0) ** 2 / 12
    radii = radii_ref[...]
    r_var = radii ** 2 / 4
    rays = rays_ref[...]
    o = rays[:, 0:3]
    dvec = rays[:, 3:6]
    dmag = jnp.maximum(1e-10, jnp.sum(dvec * dvec, axis=1, keepdims=True))
    d2 = dvec * dvec
    null = 1.0 - d2 / dmag
    means_ref[...] = jnp.concatenate(
        [dvec[:, c:c + 1] * t_mean + o[:, c:c + 1] for c in range(3)], axis=1)
    covs_ref[...] = jnp.concatenate(
        [t_var * d2[:, c:c + 1] + r_var * null[:, c:c + 1] for c in range(3)],
        axis=1)


def kernel(rays, radii, weights, t_vals):
    B = rays.shape[0]
    N = weights.shape[1]
    w2d = jnp.squeeze(weights, axis=-1)

    R1 = 512
    knots, m = pl.pallas_call(
        _prep_body,
        grid=(B // R1,),
        in_specs=[pl.BlockSpec((R1, N), lambda i: (i, 0))],
        out_specs=[pl.BlockSpec((R1, N + 1), lambda i: (i, 0)),
                   pl.BlockSpec((R1, N), lambda i: (i, 0))],
        out_shape=[jax.ShapeDtypeStruct((B, N + 1), jnp.float32),
                   jax.ShapeDtypeStruct((B, N), jnp.int32)],
    )(w2d)

    rpw = B // _NW
    knots3 = knots.reshape(_NW, rpw * (N + 1))
    t3 = t_vals.reshape(_NW, rpw * (N + 1))
    m3 = m.reshape(_NW, rpw, N).transpose(0, 2, 1).reshape(_NW, N * rpw)

    out3 = _sc_search(knots3, m3, t3)
    new_t = out3.reshape(_NW, N + 1, rpw).transpose(0, 2, 1).reshape(B, N + 1)

    R2 = 512
    means_t, covs_t = pl.pallas_call(
        _cast_body,
        grid=(B // R2,),
        in_specs=[
            pl.BlockSpec((R2, 6), lambda i: (i, 0)),
            pl.BlockSpec((R2, 1), lambda i: (i, 0)),
            pl.BlockSpec((R2, N + 1), lambda i: (i, 0)),
        ],
        out_specs=[pl.BlockSpec((R2, 3 * N), lambda i: (i, 0)),
                   pl.BlockSpec((R2, 3 * N), lambda i: (i, 0))],
        out_shape=[jax.ShapeDtypeStruct((B, 3 * N), jnp.float32),
                   jax.ShapeDtypeStruct((B, 3 * N), jnp.float32)],
    )(rays, radii, new_t)

    means = jnp.transpose(means_t.reshape(B, 3, N), (0, 2, 1))
    covs = jnp.transpose(covs_t.reshape(B, 3, N), (0, 2, 1))
    return (new_t, means, covs)
